# baseline (speedup 1.0000x reference)
import jax
import jax.numpy as jnp
from jax.experimental import pallas as pl
from jax.experimental.pallas import tpu as pltpu
from jax.experimental.pallas import tpu_sc as plsc

_RB = 8
_V = 16


def _sc_body(concat_hbm, table_hbm, out_hbm):
    rows, hidden = concat_hbm.shape
    seq_blocks = table_hbm.shape[0] // _RB

    def body(c_vmem, t_vmem, o_vmem):
        for r in range(_RB):
            @pl.loop(0, hidden, step=_V)
            def _(c):
                slc = (r, pl.ds(c, _V))
                o_vmem.at[*slc][...] = c_vmem.at[*slc][...] + t_vmem.at[*slc][...]

    pltpu.emit_pipeline(
        body,
        grid=(rows // _RB,),
        in_specs=[
            pl.BlockSpec((_RB, hidden), index_map=lambda i: (i, 0)),
            pl.BlockSpec((_RB, hidden), index_map=lambda i: (i % seq_blocks, 0)),
        ],
        out_specs=[pl.BlockSpec((_RB, hidden), index_map=lambda i: (i, 0))],
        core_axis_name=("core", "subcore"),
        dimension_semantics=(pltpu.PARALLEL,),
    )(concat_hbm, table_hbm, out_hbm)


def kernel(concat_embeddings, position_table):
    batch, seq, hidden = concat_embeddings.shape
    flat = concat_embeddings.reshape(batch * seq, hidden)
    table = position_table[:seq]

    mesh = plsc.VectorSubcoreMesh(core_axis_name="core", subcore_axis_name="subcore")
    sc_fn = pl.kernel(
        _sc_body,
        out_type=jax.ShapeDtypeStruct((batch * seq, hidden), concat_embeddings.dtype),
        mesh=mesh,
        scratch_types=[],
    )
    return sc_fn(flat, table).reshape(batch, seq, hidden)


# speedup vs baseline: 4.2144x; 4.2144x over previous
import jax
import jax.numpy as jnp
from jax.experimental import pallas as pl
from jax.experimental.pallas import tpu as pltpu

_CS = 256
_N = 4


def _body(c_hbm, t_hbm, o_hbm, in_buf, out_buf, t_vmem, in_sem, out_sem, t_sem):
    rows = c_hbm.shape[0]
    seq = t_hbm.shape[0]
    total = rows // _CS
    chunks_per_seq = seq // _CS

    def in_cp(i, slot):
        return pltpu.make_async_copy(
            c_hbm.at[pl.ds(i * _CS, _CS), :], in_buf.at[slot], in_sem.at[slot]
        )

    def out_cp(i, slot):
        return pltpu.make_async_copy(
            out_buf.at[slot], o_hbm.at[pl.ds(i * _CS, _CS), :], out_sem.at[slot]
        )

    pltpu.make_async_copy(t_hbm, t_vmem, t_sem).start()
    for j in range(min(_N, total)):
        in_cp(j, j).start()
    pltpu.make_async_copy(t_hbm, t_vmem, t_sem).wait()

    for i in range(total):
        slot = i % _N
        in_cp(i, slot).wait()
        if i >= _N:
            out_cp(i - _N, slot).wait()
        toff = (i % chunks_per_seq) * _CS
        out_buf[slot] = in_buf[slot] + t_vmem[pl.ds(toff, _CS), :]
        out_cp(i, slot).start()
        if i + _N < total:
            in_cp(i + _N, slot).start()

    for i in range(max(total - _N, 0), total):
        out_cp(i, i % _N).wait()


def kernel(concat_embeddings, position_table):
    batch, seq, hidden = concat_embeddings.shape
    flat = concat_embeddings.reshape(batch * seq, hidden)
    table = position_table[:seq]
    out = pl.pallas_call(
        _body,
        in_specs=[
            pl.BlockSpec(memory_space=pl.ANY),
            pl.BlockSpec(memory_space=pl.ANY),
        ],
        out_specs=pl.BlockSpec(memory_space=pl.ANY),
        out_shape=jax.ShapeDtypeStruct((batch * seq, hidden), concat_embeddings.dtype),
        scratch_shapes=[
            pltpu.VMEM((_N, _CS, hidden), concat_embeddings.dtype),
            pltpu.VMEM((_N, _CS, hidden), concat_embeddings.dtype),
            pltpu.VMEM((seq, hidden), concat_embeddings.dtype),
            pltpu.SemaphoreType.DMA((_N,)),
            pltpu.SemaphoreType.DMA((_N,)),
            pltpu.SemaphoreType.DMA,
        ],
    )(flat, table)
    return out.reshape(batch, seq, hidden)


# speedup vs baseline: 4.4529x; 1.0566x over previous
import jax
import jax.numpy as jnp
from jax.experimental import pallas as pl
from jax.experimental.pallas import tpu as pltpu

_CS = 256
_N = 8


def _body(c_hbm, t_hbm, o_hbm, in_buf, out_buf, t_vmem, in_sem, out_sem, t_sem):
    rows = c_hbm.shape[0]
    seq = t_hbm.shape[0]
    total = rows // _CS
    chunks_per_seq = seq // _CS

    def in_cp(i, slot):
        return pltpu.make_async_copy(
            c_hbm.at[pl.ds(i * _CS, _CS), :], in_buf.at[slot], in_sem.at[slot]
        )

    def out_cp(i, slot):
        return pltpu.make_async_copy(
            out_buf.at[slot], o_hbm.at[pl.ds(i * _CS, _CS), :], out_sem.at[slot]
        )

    pltpu.make_async_copy(t_hbm, t_vmem, t_sem).start()
    for j in range(min(_N, total)):
        in_cp(j, j).start()
    pltpu.make_async_copy(t_hbm, t_vmem, t_sem).wait()

    for i in range(total):
        slot = i % _N
        in_cp(i, slot).wait()
        if i >= _N:
            out_cp(i - _N, slot).wait()
        toff = (i % chunks_per_seq) * _CS
        out_buf[slot] = in_buf[slot] + t_vmem[pl.ds(toff, _CS), :]
        out_cp(i, slot).start()
        if i + _N < total:
            in_cp(i + _N, slot).start()

    for i in range(max(total - _N, 0), total):
        out_cp(i, i % _N).wait()


def kernel(concat_embeddings, position_table):
    batch, seq, hidden = concat_embeddings.shape
    flat = concat_embeddings.reshape(batch * seq, hidden)
    table = position_table[:seq]
    out = pl.pallas_call(
        _body,
        in_specs=[
            pl.BlockSpec(memory_space=pl.ANY),
            pl.BlockSpec(memory_space=pl.ANY),
        ],
        out_specs=pl.BlockSpec(memory_space=pl.ANY),
        out_shape=jax.ShapeDtypeStruct((batch * seq, hidden), concat_embeddings.dtype),
        scratch_shapes=[
            pltpu.VMEM((_N, _CS, hidden), concat_embeddings.dtype),
            pltpu.VMEM((_N, _CS, hidden), concat_embeddings.dtype),
            pltpu.VMEM((seq, hidden), concat_embeddings.dtype),
            pltpu.SemaphoreType.DMA((_N,)),
            pltpu.SemaphoreType.DMA((_N,)),
            pltpu.SemaphoreType.DMA,
        ],
    )(flat, table)
    return out.reshape(batch, seq, hidden)


# speedup vs baseline: 4.5456x; 1.0208x over previous
import jax
import jax.numpy as jnp
from jax.experimental import pallas as pl
from jax.experimental.pallas import tpu as pltpu

_CS = 256
_N = 16


def _body(c_hbm, t_hbm, o_hbm, in_buf, out_buf, t_vmem, in_sem, out_sem, t_sem):
    rows = c_hbm.shape[0]
    seq = t_hbm.shape[0]
    total = rows // _CS
    chunks_per_seq = seq // _CS

    def in_cp(i, slot):
        return pltpu.make_async_copy(
            c_hbm.at[pl.ds(i * _CS, _CS), :], in_buf.at[slot], in_sem.at[slot]
        )

    def out_cp(i, slot):
        return pltpu.make_async_copy(
            out_buf.at[slot], o_hbm.at[pl.ds(i * _CS, _CS), :], out_sem.at[slot]
        )

    pltpu.make_async_copy(t_hbm, t_vmem, t_sem).start()
    for j in range(min(_N, total)):
        in_cp(j, j).start()
    pltpu.make_async_copy(t_hbm, t_vmem, t_sem).wait()

    for i in range(total):
        slot = i % _N
        in_cp(i, slot).wait()
        if i >= _N:
            out_cp(i - _N, slot).wait()
        toff = (i % chunks_per_seq) * _CS
        out_buf[slot] = in_buf[slot] + t_vmem[pl.ds(toff, _CS), :]
        out_cp(i, slot).start()
        if i + _N < total:
            in_cp(i + _N, slot).start()

    for i in range(max(total - _N, 0), total):
        out_cp(i, i % _N).wait()


def kernel(concat_embeddings, position_table):
    batch, seq, hidden = concat_embeddings.shape
    flat = concat_embeddings.reshape(batch * seq, hidden)
    table = position_table[:seq]
    out = pl.pallas_call(
        _body,
        in_specs=[
            pl.BlockSpec(memory_space=pl.ANY),
            pl.BlockSpec(memory_space=pl.ANY),
        ],
        out_specs=pl.BlockSpec(memory_space=pl.ANY),
        out_shape=jax.ShapeDtypeStruct((batch * seq, hidden), concat_embeddings.dtype),
        scratch_shapes=[
            pltpu.VMEM((_N, _CS, hidden), concat_embeddings.dtype),
            pltpu.VMEM((_N, _CS, hidden), concat_embeddings.dtype),
            pltpu.VMEM((seq, hidden), concat_embeddings.dtype),
            pltpu.SemaphoreType.DMA((_N,)),
            pltpu.SemaphoreType.DMA((_N,)),
            pltpu.SemaphoreType.DMA,
        ],
    )(flat, table)
    return out.reshape(batch, seq, hidden)
